# Initial kernel scaffold; baseline (speedup 1.0000x reference)
import jax
import jax.numpy as jnp
from jax.experimental import pallas as pl


def kernel(h, ei, ea, valid_f, eps, We, be, W1, b1, W2, b2, gamma, beta):
    raise NotImplementedError("write your pallas kernel here")


# speedup vs baseline: 2.7702x; 2.7702x over previous
import functools

import jax
import jax.numpy as jnp
from jax import lax
from jax.experimental import pallas as pl
from jax.experimental.pallas import tpu as pltpu
from jax.experimental.pallas import tpu_sc as plsc

N, E, D, ED = 10000, 320000, 128, 16

NC, NS, L = 2, 16, 16
NW = NC * NS
CH = 128
NCHUNK = E // CH
N_PAD = 10240
ROWS_PER_TILE = N_PAD // NS


def _edge_emb_body(ea_ref, we_ref, be_ref, out_ref):
    out_ref[...] = (
        jnp.dot(ea_ref[...], we_ref[...], preferred_element_type=jnp.float32)
        + be_ref[...]
    )


def _edge_emb(ea, We, be2d):
    BE = 2560
    return pl.pallas_call(
        _edge_emb_body,
        grid=(E // BE,),
        in_specs=[
            pl.BlockSpec((BE, ED), lambda i: (i, 0)),
            pl.BlockSpec((ED, D), lambda i: (0, 0)),
            pl.BlockSpec((1, D), lambda i: (0, 0)),
        ],
        out_specs=pl.BlockSpec((BE, D), lambda i: (i, 0)),
        out_shape=jax.ShapeDtypeStruct((E, D), jnp.float32),
    )(ea, We, be2d)


def _sc_aggr_body(src_hbm, dst_hbm, eemb_hbm, h_hbm, out_hbm,
                  src_v, dst_v, emb_v, hrow_v, aggr_sh, sem):
    cid = lax.axis_index("c")
    sid = lax.axis_index("s")
    wid = sid * NC + cid

    def zbody(r, _):
        for j in range(D // L):
            emb_v[r, pl.ds(j * L, L)] = jnp.zeros((L,), jnp.float32)
        return 0
    lax.fori_loop(0, CH, zbody, 0)

    row0 = sid * ROWS_PER_TILE
    for k in range(ROWS_PER_TILE // CH):
        pltpu.sync_copy(emb_v, aggr_sh.at[pl.ds(row0 + k * CH, CH)])
    plsc.subcore_barrier()

    nloc = (NCHUNK - wid + NW - 1) // NW

    def chunk_body(k, _):
        base = (wid + k * NW) * CH
        pltpu.sync_copy(src_hbm.at[pl.ds(base, CH)], src_v)
        pltpu.sync_copy(dst_hbm.at[pl.ds(base, CH)], dst_v)
        pltpu.sync_copy(eemb_hbm.at[pl.ds(base, CH)], emb_v)
        pltpu.async_copy(h_hbm.at[src_v], hrow_v, sem).wait()

        def rbody(r, _):
            for j in range(D // L):
                sl = pl.ds(j * L, L)
                emb_v[r, sl] = jnp.maximum(emb_v[r, sl] + hrow_v[r, sl], 0.0)
            return 0
        lax.fori_loop(0, CH, rbody, 0)

        pltpu.sync_copy(emb_v, aggr_sh.at[dst_v], add=True)
        return 0

    lax.fori_loop(0, nloc, chunk_body, 0)
    plsc.subcore_barrier()

    pltpu.sync_copy(aggr_sh.at[pl.ds(row0, ROWS_PER_TILE)],
                    out_hbm.at[cid, pl.ds(row0, ROWS_PER_TILE)])


def _sc_aggregate(src, dst, e_emb, h):
    mesh = plsc.VectorSubcoreMesh(core_axis_name="c", subcore_axis_name="s",
                                  num_cores=NC, num_subcores=NS)
    fn = pl.kernel(
        _sc_aggr_body,
        out_type=jax.ShapeDtypeStruct((NC, N_PAD, D), jnp.float32),
        mesh=mesh,
        scratch_types=[
            pltpu.VMEM((CH,), jnp.int32),
            pltpu.VMEM((CH,), jnp.int32),
            pltpu.VMEM((CH, D), jnp.float32),
            pltpu.VMEM((CH, D), jnp.float32),
            pltpu.VMEM_SHARED((N_PAD, D), jnp.float32),
            pltpu.SemaphoreType.DMA,
        ],
    )
    return fn(src, dst, e_emb, h)


def _node_body(h_ref, a_ref, eps_ref, w1_ref, b1_ref, w2_ref, b2_ref,
               g_ref, bt_ref, vf_ref, out_ref):
    h = h_ref[...]
    aggr = a_ref[0, :N] + a_ref[1, :N]
    z = (1.0 + eps_ref[...]) * h + aggr
    z = jnp.maximum(
        jnp.dot(z, w1_ref[...], preferred_element_type=jnp.float32)
        + b1_ref[...], 0.0)
    z = (jnp.dot(z, w2_ref[...], preferred_element_type=jnp.float32)
         + b2_ref[...])
    z = jnp.maximum(z, 0.0)
    mean = jnp.mean(z, axis=0, keepdims=True)
    var = jnp.mean(jnp.square(z - mean), axis=0, keepdims=True)
    z = g_ref[...] * (z - mean) * lax.rsqrt(var + 1e-5) + bt_ref[...]
    out_ref[...] = (h + z) * vf_ref[...]


def _node_update(h, aggr2, eps2d, W1, b1_2d, W2, b2_2d, g2d, bt2d, valid_f):
    return pl.pallas_call(
        _node_body,
        out_shape=jax.ShapeDtypeStruct((N, D), jnp.float32),
    )(h, aggr2, eps2d, W1, b1_2d, W2, b2_2d, g2d, bt2d, valid_f)


@jax.jit
def kernel(h, ei, ea, valid_f, eps, We, be, W1, b1, W2, b2, gamma, beta):
    e_emb = _edge_emb(ea, We, be.reshape(1, D))
    src = ei[0]
    dst = ei[1]
    aggr2 = _sc_aggregate(src, dst, e_emb, h)
    return _node_update(
        h, aggr2, eps.reshape(1, 1), W1, b1.reshape(1, D), W2,
        b2.reshape(1, D), gamma.reshape(1, D), beta.reshape(1, D), valid_f)
